# Initial kernel scaffold; baseline (speedup 1.0000x reference)
import jax
import jax.numpy as jnp
from jax.experimental import pallas as pl


def kernel(x, edge_list, W1l, b1, W1r, W2l, b2, W2r, Wlin1, blin1, Wlin2, blin2):
    raise NotImplementedError("write your pallas kernel here")


# speedup vs baseline: 10.1935x; 10.1935x over previous
import functools

import jax
import jax.numpy as jnp
from jax import lax
from jax.experimental import pallas as pl
from jax.experimental.pallas import tpu as pltpu
from jax.experimental.pallas import tpu_sc as plsc

NN = 10000
EE = 320000
FH = 16
NC = 2
NS = 16
NW = NC * NS
EPT = EE // NW
CH = 80
NCHK = EPT // CH
NIO = 10
RPS = NN // NIO


def _sc_agg(with_cnt):
    mesh = plsc.VectorSubcoreMesh(core_axis_name="c", subcore_axis_name="s")
    out_type = [jax.ShapeDtypeStruct((NC, NN, FH), jnp.float32)]
    scratch = [
        pltpu.VMEM((NCHK, CH), jnp.int32),
        pltpu.VMEM((NCHK, CH), jnp.int32),
        pltpu.VMEM((CH, FH), jnp.float32),
        pltpu.VMEM((RPS, FH), jnp.float32),
        pltpu.VMEM_SHARED((NN, FH), jnp.float32),
        pltpu.SemaphoreType.DMA,
    ]
    if with_cnt:
        out_type.append(jax.ShapeDtypeStruct((NC, NN, FH), jnp.float32))
        scratch.append(pltpu.VMEM((CH, FH), jnp.float32))
        scratch.append(pltpu.VMEM_SHARED((NN, FH), jnp.float32))

    def body(p_hbm, src_hbm, dst_hbm, acc_out, *rest):
        if with_cnt:
            cnt_out, src_v, dst_v, rows, zbuf, acc_sh, sem, ones, cnt_sh = rest
        else:
            src_v, dst_v, rows, zbuf, acc_sh, sem = rest
        c = lax.axis_index("c")
        s = lax.axis_index("s")
        wid = c * NS + s

        pltpu.sync_copy(src_hbm.at[wid], src_v)
        pltpu.sync_copy(dst_hbm.at[wid], dst_v)

        def zrow(i, carry):
            zbuf[i, :] = jnp.zeros((FH,), jnp.float32)
            return carry
        lax.fori_loop(0, RPS, zrow, 0)

        @pl.when(s < NIO)
        def _init():
            pltpu.sync_copy(zbuf, acc_sh.at[pl.ds(s * RPS, RPS)])
            if with_cnt:
                pltpu.sync_copy(zbuf, cnt_sh.at[pl.ds(s * RPS, RPS)])
        if with_cnt:
            def orow(i, carry):
                ones[i, :] = jnp.ones((FH,), jnp.float32)
                return carry
            lax.fori_loop(0, CH, orow, 0)
        plsc.subcore_barrier()

        def step(i, carry):
            pltpu.async_copy(p_hbm.at[src_v.at[i]], rows, sem).wait()
            pltpu.sync_copy(rows, acc_sh.at[dst_v.at[i]], add=True)
            if with_cnt:
                pltpu.sync_copy(ones, cnt_sh.at[dst_v.at[i]], add=True)
            return carry
        lax.fori_loop(0, NCHK, step, 0)

        plsc.subcore_barrier()

        @pl.when(s < NIO)
        def _writeout():
            row0 = s * RPS
            pltpu.sync_copy(acc_sh.at[pl.ds(row0, RPS)],
                            acc_out.at[c, pl.ds(row0, RPS)])
            if with_cnt:
                pltpu.sync_copy(cnt_sh.at[pl.ds(row0, RPS)],
                                cnt_out.at[c, pl.ds(row0, RPS)])

    return pl.kernel(body, out_type=out_type, mesh=mesh,
                     scratch_types=scratch,
                     compiler_params=pltpu.CompilerParams(
                         use_tc_tiling_on_sc=False))


def _tc_proj1(x_ref, wl_ref, wr_ref, p_ref, r_ref):
    x = x_ref[...]
    p_ref[...] = jnp.dot(x, wl_ref[...], preferred_element_type=jnp.float32)
    r_ref[...] = jnp.dot(x, wr_ref[...], preferred_element_type=jnp.float32)


def _tc_combine1(acc_ref, cnt_ref, r1_ref, b1_ref, wl_ref, wr_ref,
                 p2_ref, r2_ref, inv_ref):
    cnt = cnt_ref[0] + cnt_ref[1]
    inv = 1.0 / jnp.maximum(cnt, 1.0)
    mean = (acc_ref[0] + acc_ref[1]) * inv
    h1 = jax.nn.sigmoid(mean + b1_ref[...] + r1_ref[...])
    p2_ref[...] = jnp.dot(h1, wl_ref[...], preferred_element_type=jnp.float32)
    r2_ref[...] = jnp.dot(h1, wr_ref[...], preferred_element_type=jnp.float32)
    inv_ref[...] = inv


def _tc_combine2(acc_ref, inv_ref, r2_ref, b2_ref, wlin1_ref, blin1_ref,
                 wlin2_ref, blin2_ref, out_ref):
    mean = (acc_ref[0] + acc_ref[1]) * inv_ref[...]
    h2 = jax.nn.sigmoid(mean + b2_ref[...] + r2_ref[...])
    h3 = jax.nn.sigmoid(
        jnp.dot(h2, wlin1_ref[...], preferred_element_type=jnp.float32)
        + blin1_ref[...])
    out_ref[...] = (
        jnp.dot(h3, wlin2_ref[...], preferred_element_type=jnp.float32)
        + blin2_ref[...])


@jax.jit
def kernel(x, edge_list, W1l, b1, W1r, W2l, b2, W2r, Wlin1, blin1, Wlin2, blin2):
    f32 = jnp.float32
    src = edge_list[0].astype(jnp.int32).reshape(NW, NCHK, CH)
    dst = edge_list[1].astype(jnp.int32).reshape(NW, NCHK, CH)

    sds = jax.ShapeDtypeStruct
    p1, r1 = pl.pallas_call(
        _tc_proj1,
        out_shape=[sds((NN, FH), f32), sds((NN, FH), f32)],
    )(x, W1l.T, W1r.T)

    accp1, cntp1 = _sc_agg(True)(p1, src, dst)

    p2, r2, inv = pl.pallas_call(
        _tc_combine1,
        out_shape=[sds((NN, FH), f32), sds((NN, FH), f32), sds((NN, FH), f32)],
    )(accp1, cntp1, r1, b1.reshape(1, FH), W2l.T, W2r.T)

    (accp2,) = _sc_agg(False)(p2, src, dst)

    out = pl.pallas_call(
        _tc_combine2,
        out_shape=sds((NN, FH), f32),
    )(accp2, inv, r2, b2.reshape(1, FH), Wlin1.T, blin1.reshape(1, FH),
      Wlin2.T, blin2.reshape(1, FH))
    return out


# speedup vs baseline: 14.7319x; 1.4452x over previous
import functools

import jax
import jax.numpy as jnp
from jax import lax
from jax.experimental import pallas as pl
from jax.experimental.pallas import tpu as pltpu
from jax.experimental.pallas import tpu_sc as plsc

NN = 10000
EE = 320000
FH = 16
NC = 2
NS = 16
NW = NC * NS
EPT = EE // NW
CH = 80
NCHK = EPT // CH
NIO = 10
RPS = NN // NIO


def _sc_agg(with_cnt):
    mesh = plsc.VectorSubcoreMesh(core_axis_name="c", subcore_axis_name="s")
    out_type = [jax.ShapeDtypeStruct((NC, NN, FH), jnp.float32)]
    scratch = [
        pltpu.VMEM((NCHK, CH), jnp.int32),
        pltpu.VMEM((NCHK, CH), jnp.int32),
        pltpu.VMEM((CH, FH), jnp.float32),
        pltpu.VMEM((CH, FH), jnp.float32),
        pltpu.VMEM((RPS, FH), jnp.float32),
        pltpu.VMEM_SHARED((NN, FH), jnp.float32),
        pltpu.SemaphoreType.DMA,
        pltpu.SemaphoreType.DMA,
    ]
    if with_cnt:
        out_type.append(jax.ShapeDtypeStruct((NC, NN, FH), jnp.float32))
        scratch.append(pltpu.VMEM((CH, FH), jnp.float32))
        scratch.append(pltpu.VMEM_SHARED((NN, FH), jnp.float32))

    def body(p_hbm, src_hbm, dst_hbm, acc_out, *rest):
        if with_cnt:
            (cnt_out, src_v, dst_v, rows0, rows1, zbuf, acc_sh, sem0, sem1,
             ones, cnt_sh) = rest
        else:
            src_v, dst_v, rows0, rows1, zbuf, acc_sh, sem0, sem1 = rest
        c = lax.axis_index("c")
        s = lax.axis_index("s")
        wid = c * NS + s

        pltpu.sync_copy(src_hbm.at[wid], src_v)
        pltpu.sync_copy(dst_hbm.at[wid], dst_v)

        def zrow(i, carry):
            zbuf[i, :] = jnp.zeros((FH,), jnp.float32)
            return carry
        lax.fori_loop(0, RPS, zrow, 0)

        @pl.when(s < NIO)
        def _init():
            pltpu.sync_copy(zbuf, acc_sh.at[pl.ds(s * RPS, RPS)])
            if with_cnt:
                pltpu.sync_copy(zbuf, cnt_sh.at[pl.ds(s * RPS, RPS)])
        if with_cnt:
            def orow(i, carry):
                ones[i, :] = jnp.ones((FH,), jnp.float32)
                return carry
            lax.fori_loop(0, CH, orow, 0)
        plsc.subcore_barrier()

        def drain(rows, sem):
            pltpu.make_async_copy(p_hbm.at[pl.ds(0, CH)], rows, sem).wait()

        def scat(rows, i):
            pltpu.sync_copy(rows, acc_sh.at[dst_v.at[i]], add=True)
            if with_cnt:
                pltpu.sync_copy(ones, cnt_sh.at[dst_v.at[i]], add=True)

        pltpu.async_copy(p_hbm.at[src_v.at[0]], rows0, sem0)

        def step(j, carry):
            i0 = 2 * j
            pltpu.async_copy(p_hbm.at[src_v.at[i0 + 1]], rows1, sem1)
            drain(rows0, sem0)
            scat(rows0, i0)
            pltpu.async_copy(p_hbm.at[src_v.at[i0 + 2]], rows0, sem0)
            drain(rows1, sem1)
            scat(rows1, i0 + 1)
            return carry
        lax.fori_loop(0, NCHK // 2, step, 0)

        drain(rows0, sem0)
        scat(rows0, NCHK - 1)

        plsc.subcore_barrier()

        @pl.when(s < NIO)
        def _writeout():
            row0 = s * RPS
            pltpu.sync_copy(acc_sh.at[pl.ds(row0, RPS)],
                            acc_out.at[c, pl.ds(row0, RPS)])
            if with_cnt:
                pltpu.sync_copy(cnt_sh.at[pl.ds(row0, RPS)],
                                cnt_out.at[c, pl.ds(row0, RPS)])

    return pl.kernel(body, out_type=out_type, mesh=mesh,
                     scratch_types=scratch,
                     compiler_params=pltpu.CompilerParams(
                         use_tc_tiling_on_sc=False))


def _tc_proj1(x_ref, wl_ref, wr_ref, p_ref, r_ref):
    x = x_ref[...]
    p_ref[...] = jnp.dot(x, wl_ref[...], preferred_element_type=jnp.float32)
    r_ref[...] = jnp.dot(x, wr_ref[...], preferred_element_type=jnp.float32)


def _tc_combine1(acc_ref, cnt_ref, r1_ref, b1_ref, wl_ref, wr_ref,
                 p2_ref, r2_ref, inv_ref):
    cnt = cnt_ref[0] + cnt_ref[1]
    inv = 1.0 / jnp.maximum(cnt, 1.0)
    mean = (acc_ref[0] + acc_ref[1]) * inv
    h1 = jax.nn.sigmoid(mean + b1_ref[...] + r1_ref[...])
    p2_ref[...] = jnp.dot(h1, wl_ref[...], preferred_element_type=jnp.float32)
    r2_ref[...] = jnp.dot(h1, wr_ref[...], preferred_element_type=jnp.float32)
    inv_ref[...] = inv


def _tc_combine2(acc_ref, inv_ref, r2_ref, b2_ref, wlin1_ref, blin1_ref,
                 wlin2_ref, blin2_ref, out_ref):
    mean = (acc_ref[0] + acc_ref[1]) * inv_ref[...]
    h2 = jax.nn.sigmoid(mean + b2_ref[...] + r2_ref[...])
    h3 = jax.nn.sigmoid(
        jnp.dot(h2, wlin1_ref[...], preferred_element_type=jnp.float32)
        + blin1_ref[...])
    out_ref[...] = (
        jnp.dot(h3, wlin2_ref[...], preferred_element_type=jnp.float32)
        + blin2_ref[...])


@jax.jit
def kernel(x, edge_list, W1l, b1, W1r, W2l, b2, W2r, Wlin1, blin1, Wlin2, blin2):
    f32 = jnp.float32
    src = edge_list[0].astype(jnp.int32).reshape(NW, NCHK, CH)
    dst = edge_list[1].astype(jnp.int32).reshape(NW, NCHK, CH)

    sds = jax.ShapeDtypeStruct
    p1, r1 = pl.pallas_call(
        _tc_proj1,
        out_shape=[sds((NN, FH), f32), sds((NN, FH), f32)],
    )(x, W1l.T, W1r.T)

    accp1, cntp1 = _sc_agg(True)(p1, src, dst)

    p2, r2, inv = pl.pallas_call(
        _tc_combine1,
        out_shape=[sds((NN, FH), f32), sds((NN, FH), f32), sds((NN, FH), f32)],
    )(accp1, cntp1, r1, b1.reshape(1, FH), W2l.T, W2r.T)

    (accp2,) = _sc_agg(False)(p2, src, dst)

    out = pl.pallas_call(
        _tc_combine2,
        out_shape=sds((NN, FH), f32),
    )(accp2, inv, r2, b2.reshape(1, FH), Wlin1.T, blin1.reshape(1, FH),
      Wlin2.T, blin2.reshape(1, FH))
    return out
